# baseline (speedup 1.0000x reference)
import functools

import jax
import jax.numpy as jnp
from jax import lax
from jax.experimental import pallas as pl
from jax.experimental.pallas import tpu as pltpu
from jax.experimental.pallas import tpu_sc as plsc

_NUM_NODES = 1000000
_EMBED_DIM = 64
_NUM_CORES = 2
_NUM_SUBCORES = 16
_NUM_WORKERS = _NUM_CORES * _NUM_SUBCORES
_ROWS_PER_W = (_NUM_NODES // _NUM_WORKERS) // 8 * 8
_TAIL_BASE = _ROWS_PER_W * _NUM_WORKERS
_TAIL_ROWS = _NUM_NODES - _TAIL_BASE

_MESH = plsc.VectorSubcoreMesh(core_axis_name="c", subcore_axis_name="s")


@functools.partial(
    pl.kernel,
    out_type=jax.ShapeDtypeStruct((_NUM_NODES, _EMBED_DIM), jnp.float32),
    mesh=_MESH,
    scratch_types=[pltpu.SemaphoreType.DMA],
)
def _sc_copy(w_hbm, o_hbm, sem):
    wid = lax.axis_index("s") * _NUM_CORES + lax.axis_index("c")
    base = pl.multiple_of(wid * _ROWS_PER_W, 8)
    pltpu.make_async_copy(
        w_hbm.at[pl.ds(base, _ROWS_PER_W)],
        o_hbm.at[pl.ds(base, _ROWS_PER_W)],
        sem,
    ).start()

    @pl.when(wid == 0)
    def _():
        pltpu.make_async_copy(
            w_hbm.at[pl.ds(_TAIL_BASE, _TAIL_ROWS)],
            o_hbm.at[pl.ds(_TAIL_BASE, _TAIL_ROWS)],
            sem,
        ).start()
        pltpu.make_async_copy(
            w_hbm.at[pl.ds(_TAIL_BASE, _TAIL_ROWS)],
            o_hbm.at[pl.ds(_TAIL_BASE, _TAIL_ROWS)],
            sem,
        ).wait()

    pltpu.make_async_copy(
        w_hbm.at[pl.ds(base, _ROWS_PER_W)],
        o_hbm.at[pl.ds(base, _ROWS_PER_W)],
        sem,
    ).wait()


def kernel(weight):
    return _sc_copy(weight)


# speedup vs baseline: 15.2916x; 15.2916x over previous
import functools

import jax
import jax.numpy as jnp
from jax import lax
from jax.experimental import pallas as pl
from jax.experimental.pallas import tpu as pltpu
from jax.experimental.pallas import tpu_sc as plsc

_NUM_NODES = 1000000
_EMBED_DIM = 64
_NUM_CORES = 2
_NUM_SUBCORES = 16
_NUM_WORKERS = _NUM_CORES * _NUM_SUBCORES
_ROWS_PER_W = (_NUM_NODES // _NUM_WORKERS) // 8 * 8
_TAIL_BASE = _ROWS_PER_W * _NUM_WORKERS
_TAIL_ROWS = _NUM_NODES - _TAIL_BASE

_NBUF = 3
_CHUNK = 248
_NCHUNKS = _ROWS_PER_W // _CHUNK
_NGROUPS = _NCHUNKS // _NBUF
assert _CHUNK * _NCHUNKS == _ROWS_PER_W and _NBUF * _NGROUPS == _NCHUNKS

_MESH = plsc.VectorSubcoreMesh(core_axis_name="c", subcore_axis_name="s")


@functools.partial(
    pl.kernel,
    out_type=jax.ShapeDtypeStruct((_NUM_NODES, _EMBED_DIM), jnp.float32),
    mesh=_MESH,
    scratch_types=[
        [pltpu.VMEM((_CHUNK, _EMBED_DIM), jnp.float32) for _ in range(_NBUF)],
        [pltpu.SemaphoreType.DMA for _ in range(_NBUF)],
        [pltpu.SemaphoreType.DMA for _ in range(_NBUF)],
    ],
)
def _sc_copy(w_hbm, o_hbm, bufs, in_sems, out_sems):
    wid = lax.axis_index("s") * _NUM_CORES + lax.axis_index("c")
    base = pl.multiple_of(wid * _ROWS_PER_W, 8)

    def _in_copy(k, b):
        off = pl.multiple_of(base + k * _CHUNK, 8)
        return pltpu.make_async_copy(
            w_hbm.at[pl.ds(off, _CHUNK)], bufs[b], in_sems[b])

    def _out_copy(k, b):
        off = pl.multiple_of(base + k * _CHUNK, 8)
        return pltpu.make_async_copy(
            bufs[b], o_hbm.at[pl.ds(off, _CHUNK)], out_sems[b])

    for j in range(_NBUF):
        _in_copy(j, j).start()

    def _group(g, carry):
        for j in range(_NBUF):
            k = g * _NBUF + j
            _in_copy(k, j).wait()
            _out_copy(k, j).start()
        for j in range(_NBUF):
            k = g * _NBUF + j

            @pl.when(k + _NBUF < _NCHUNKS)
            def _():
                _out_copy(k, j).wait()
                _in_copy(k + _NBUF, j).start()

        return carry

    lax.fori_loop(0, _NGROUPS, _group, 0)

    for j in range(_NBUF):
        _out_copy(_NCHUNKS - _NBUF + j, j).wait()

    @pl.when(wid == 0)
    def _():
        pltpu.make_async_copy(
            w_hbm.at[pl.ds(_TAIL_BASE, _TAIL_ROWS)],
            bufs[0].at[pl.ds(0, _TAIL_ROWS)], in_sems[0]).start()
        pltpu.make_async_copy(
            w_hbm.at[pl.ds(_TAIL_BASE, _TAIL_ROWS)],
            bufs[0].at[pl.ds(0, _TAIL_ROWS)], in_sems[0]).wait()
        pltpu.make_async_copy(
            bufs[0].at[pl.ds(0, _TAIL_ROWS)],
            o_hbm.at[pl.ds(_TAIL_BASE, _TAIL_ROWS)], out_sems[0]).start()
        pltpu.make_async_copy(
            bufs[0].at[pl.ds(0, _TAIL_ROWS)],
            o_hbm.at[pl.ds(_TAIL_BASE, _TAIL_ROWS)], out_sems[0]).wait()


def kernel(weight):
    return _sc_copy(weight)
